# baseline (speedup 1.0000x reference)
import jax
import jax.numpy as jnp
from jax import lax
from jax.experimental import pallas as pl
from jax.experimental.pallas import tpu as pltpu
from jax.experimental.pallas import tpu_sc as plsc

_NCORES = 2
_NSUB = 16
_NWORKERS = _NCORES * _NSUB
_NSLOTS = 6
_LOOKAHEAD = 4
_K = 4
_SLAB = 56
_TC_GROUP = 16


def _sc_gather_chunk(k, chunk_batches, seq, dim, table, idx32):
    per_worker = chunk_batches // _NWORKERS
    mesh = plsc.VectorSubcoreMesh(core_axis_name="c", subcore_axis_name="s")
    sem_types = [pltpu.SemaphoreType.DMA] * (2 * _NSLOTS)

    @pl.kernel(
        out_type=jax.ShapeDtypeStruct((chunk_batches, _SLAB, dim),
                                      table.dtype),
        mesh=mesh,
        scratch_types=[
            pltpu.VMEM((per_worker, seq), jnp.int32),
            pltpu.VMEM((_NSLOTS, _SLAB, dim), table.dtype),
        ] + sem_types,
    )
    def gather_kernel(table_hbm, idx_hbm, out_hbm, idx_v, rows_v, *sems):
        g_sems = sems[:_NSLOTS]
        s_sems = sems[_NSLOTS:]
        wid = lax.axis_index("s") * _NCORES + lax.axis_index("c")
        base = wid * per_worker

        pltpu.sync_copy(
            idx_hbm.at[pl.ds(k * chunk_batches + base, per_worker)], idx_v)

        def fire(c, slot):
            pltpu.async_copy(
                table_hbm.at[idx_v.at[c]],
                rows_v.at[slot, pl.ds(0, seq)], g_sems[slot])

        def wait_gather(slot):
            pltpu.make_async_copy(
                table_hbm.at[idx_v.at[0]],
                rows_v.at[slot, pl.ds(0, seq)], g_sems[slot]).wait()

        def store(c, slot):
            pltpu.async_copy(
                rows_v.at[slot], out_hbm.at[base + c], s_sems[slot])

        def wait_store(slot):
            pltpu.make_async_copy(
                rows_v.at[slot], out_hbm.at[base], s_sems[slot]).wait()

        for c in range(_LOOKAHEAD):
            fire(c, c % _NSLOTS)

        def chunk_body(c, slot, fire_next, wait_prev_store):
            wait_gather(slot)
            store(c, slot)
            if fire_next:
                nxt_slot = (slot + _LOOKAHEAD) % _NSLOTS
                if wait_prev_store:
                    wait_store(nxt_slot)
                fire(c + _LOOKAHEAD, nxt_slot)

        chunk_body(0, 0, True, False)
        chunk_body(1, 1, True, False)

        steady = ((per_worker - _LOOKAHEAD - 2) // _NSLOTS) * _NSLOTS

        @pl.loop(2, 2 + steady, step=_NSLOTS)
        def _(c0):
            for j in range(_NSLOTS):
                chunk_body(c0 + j, (2 + j) % _NSLOTS, True, True)

        c = 2 + steady
        while c + _LOOKAHEAD < per_worker:
            chunk_body(c, c % _NSLOTS, True, True)
            c += 1
        while c < per_worker:
            chunk_body(c, c % _NSLOTS, False, False)
            c += 1

        for s in range(_NSLOTS):
            wait_store(s)

    return gather_kernel(table, idx32)


def _tc_slice_chunk(k, chunk_batches, seq, dim, slabs, carry, batch, first):
    grid = chunk_batches // _TC_GROUP

    def body(*refs):
        if first:
            in_ref, o_ref = refs
        else:
            in_ref, _, o_ref = refs
        o_ref[...] = in_ref[:, :seq, :]

    in_specs = [pl.BlockSpec((_TC_GROUP, _SLAB, dim), lambda i: (i, 0, 0))]
    operands = (slabs,)
    kwargs = {}
    if not first:
        in_specs.append(pl.BlockSpec(memory_space=pl.ANY))
        operands = (slabs, carry)
        kwargs["input_output_aliases"] = {1: 0}

    return pl.pallas_call(
        body,
        grid=(grid,),
        in_specs=in_specs,
        out_specs=pl.BlockSpec(
            (_TC_GROUP, seq, dim),
            lambda i, _k=k, _g=grid: (_k * _g + i, 0, 0)),
        out_shape=jax.ShapeDtypeStruct((batch, seq, dim), slabs.dtype),
        **kwargs,
    )(*operands)


def kernel(indices, table):
    batch, seq = indices.shape
    num_rows, dim = table.shape
    idx32 = indices.astype(jnp.int32)
    chunk_batches = batch // _K

    out = None
    for k in range(_K):
        slabs = _sc_gather_chunk(k, chunk_batches, seq, dim, table, idx32)
        out = _tc_slice_chunk(k, chunk_batches, seq, dim, slabs, out,
                              batch, first=(k == 0))
    return out


# speedup vs baseline: 1.9915x; 1.9915x over previous
import jax
import jax.numpy as jnp
from jax import lax
from jax.experimental import pallas as pl
from jax.experimental.pallas import tpu as pltpu
from jax.experimental.pallas import tpu_sc as plsc

_NCORES = 2
_NSUB = 16
_NWORKERS = _NCORES * _NSUB
_NSLOTS = 10
_LOOKAHEAD = 8


def kernel(indices, table):
    batch, seq = indices.shape
    num_rows, dim = table.shape
    per_worker = batch // _NWORKERS
    idx32 = indices.astype(jnp.int32)

    mesh = plsc.VectorSubcoreMesh(core_axis_name="c", subcore_axis_name="s")

    sem_types = [pltpu.SemaphoreType.DMA] * (2 * _NSLOTS)

    @pl.kernel(
        out_type=jax.ShapeDtypeStruct((batch, seq, dim), table.dtype),
        mesh=mesh,
        scratch_types=[
            pltpu.VMEM((per_worker, seq), jnp.int32),
            pltpu.VMEM((_NSLOTS, seq, dim), table.dtype),
        ] + sem_types,
    )
    def gather_kernel(table_hbm, idx_hbm, out_hbm, idx_v, rows_v, *sems):
        g_sems = sems[:_NSLOTS]
        s_sems = sems[_NSLOTS:]
        wid = lax.axis_index("s") * _NCORES + lax.axis_index("c")
        base = wid * per_worker

        pltpu.sync_copy(idx_hbm.at[pl.ds(base, per_worker)], idx_v)

        def fire(c, slot):
            pltpu.async_copy(
                table_hbm.at[idx_v.at[c]], rows_v.at[slot], g_sems[slot])

        def wait_gather(slot):
            pltpu.make_async_copy(
                table_hbm.at[idx_v.at[0]],
                rows_v.at[slot], g_sems[slot]).wait()

        def store(c, slot):
            pltpu.async_copy(
                rows_v.at[slot], out_hbm.at[base + c], s_sems[slot])

        def wait_store(slot):
            pltpu.make_async_copy(
                rows_v.at[slot], out_hbm.at[base], s_sems[slot]).wait()

        for c in range(_LOOKAHEAD):
            fire(c, c % _NSLOTS)

        def chunk_body(c, slot, fire_next, wait_prev_store):
            wait_gather(slot)
            store(c, slot)
            if fire_next:
                nxt_slot = (slot + _LOOKAHEAD) % _NSLOTS
                if wait_prev_store:
                    wait_store(nxt_slot)
                fire(c + _LOOKAHEAD, nxt_slot)

        chunk_body(0, 0, True, False)
        chunk_body(1, 1, True, False)

        steady = ((per_worker - _LOOKAHEAD - 2) // _NSLOTS) * _NSLOTS

        @pl.loop(2, 2 + steady, step=_NSLOTS)
        def _(c0):
            for j in range(_NSLOTS):
                chunk_body(c0 + j, (2 + j) % _NSLOTS, True, True)

        c = 2 + steady
        while c + _LOOKAHEAD < per_worker:
            chunk_body(c, c % _NSLOTS, True, True)
            c += 1
        while c < per_worker:
            chunk_body(c, c % _NSLOTS, False, False)
            c += 1

        for s in range(_NSLOTS):
            wait_store(s)

    return gather_kernel(table, idx32)
